# baseline (speedup 1.0000x reference)
import functools

import jax
import jax.numpy as jnp
from jax import lax
from jax.experimental import pallas as pl
from jax.experimental.pallas import tpu as pltpu
from jax.experimental.pallas import tpu_sc as plsc

N = 10000
E = 320000
D = 128
HEADS = 2
OUT = 128
HO = HEADS * OUT

NC = 2
NS = 16
NW = NC * NS

EPT1 = E // NW
CH1 = 80
NCH1 = EPT1 // CH1
EPT2 = E // NS
BB = 800
CH3 = 80
DENP = 40960
RB = N // NS
NEG_SLOPE = 0.2

_mesh = plsc.VectorSubcoreMesh(core_axis_name="c", subcore_axis_name="s")


def _mm_body(x_ref, wh_ref, waa_ref, hcat_ref, aa_ref):
    xb = x_ref[...]
    hcat_ref[...] = jnp.dot(xb, wh_ref[...], preferred_element_type=jnp.float32)
    aa_ref[...] = jnp.dot(xb, waa_ref[...], preferred_element_type=jnp.float32)


def _matmul(x, W, Waa):
    bs = 2000
    nb = N // bs
    return pl.pallas_call(
        _mm_body,
        grid=(2 * nb,),
        in_specs=[
            pl.BlockSpec((bs, D), lambda i: (i % nb, 0)),
            pl.BlockSpec((D, OUT), lambda i: (0, i // nb)),
            pl.BlockSpec((D, 4), lambda i: (0, 0)),
        ],
        out_specs=[
            pl.BlockSpec((bs, OUT), lambda i: (i, 0)),
            pl.BlockSpec((bs, 4), lambda i: (i % nb, 0)),
        ],
        out_shape=[
            jax.ShapeDtypeStruct((2 * N, OUT), jnp.float32),
            jax.ShapeDtypeStruct((N, 4), jnp.float32),
        ],
    )(x, W, Waa)


def _s1_body(edge_f, perm_h, aa_h, zf_h,
             ppos_o, pneg_o, s2_o, den_o,
             perm_v, aa_v, sb, db, s2b, p4, vidx, den_acc):
    ci = lax.axis_index("c")
    si = lax.axis_index("s")
    wid = ci * NS + si

    pltpu.sync_copy(perm_h, perm_v)
    pltpu.sync_copy(aa_h, aa_v)
    pltpu.sync_copy(zf_h, den_acc.at[pl.ds(si * (DENP // NS), DENP // NS)])
    plsc.subcore_barrier()

    c0 = jnp.zeros((16,), jnp.int32)
    c1 = jnp.full((16,), 1, jnp.int32)
    c2 = jnp.full((16,), 2, jnp.int32)
    c3 = jnp.full((16,), 3, jnp.int32)

    def lrelu_exp(a, b):
        e = a + b
        return jnp.exp(jnp.where(e > 0, e, NEG_SLOPE * e))

    def chunk(c, _):
        base = wid * EPT1 + c * CH1
        pltpu.sync_copy(edge_f.at[pl.ds(base, CH1)], sb)
        pltpu.sync_copy(edge_f.at[pl.ds(E + base, CH1)], db)
        for g in range(CH1 // 16):
            sl = pl.ds(g * 16, 16)
            s = sb[sl]
            d = db[sl]
            s2 = plsc.load_gather(perm_v, [s])
            d2 = plsc.load_gather(perm_v, [d])
            p0 = lrelu_exp(plsc.load_gather(aa_v, [s, c0]),
                           plsc.load_gather(aa_v, [d, c2]))
            p1 = lrelu_exp(plsc.load_gather(aa_v, [s, c1]),
                           plsc.load_gather(aa_v, [d, c3]))
            q0 = lrelu_exp(plsc.load_gather(aa_v, [s2, c0]),
                           plsc.load_gather(aa_v, [d2, c2]))
            q1 = lrelu_exp(plsc.load_gather(aa_v, [s2, c1]),
                           plsc.load_gather(aa_v, [d2, c3]))
            s2b[sl] = s2
            p4[0, sl] = p0
            p4[1, sl] = p1
            p4[2, sl] = q0
            p4[3, sl] = q1
            vidx[0, sl] = d
            vidx[1, sl] = d + N
            vidx[2, sl] = d + 2 * N
            vidx[3, sl] = d + 3 * N
        pltpu.sync_copy(s2b, s2_o.at[pl.ds(base, CH1)])
        for j in range(4):
            pltpu.sync_copy(p4.at[j], (ppos_o if j < 2 else pneg_o).at[
                pl.ds((j % 2) * E + base, CH1)])
        for j in range(4):
            pltpu.sync_copy(p4.at[j], den_acc.at[vidx.at[j]], add=True)
        return 0

    lax.fori_loop(0, NCH1, chunk, 0)
    plsc.subcore_barrier()
    @pl.when(si < 8)
    def _():
        pltpu.sync_copy(den_acc.at[pl.ds(si * 5120, 5120)],
                        den_o.at[pl.ds(ci * DENP + si * 5120, 5120)])


_stage1 = functools.partial(
    pl.kernel,
    out_type=[
        jax.ShapeDtypeStruct((2 * E,), jnp.float32),
        jax.ShapeDtypeStruct((2 * E,), jnp.float32),
        jax.ShapeDtypeStruct((E,), jnp.int32),
        jax.ShapeDtypeStruct((NC * DENP,), jnp.float32),
    ],
    mesh=_mesh,
    scratch_types=[
        pltpu.VMEM((N,), jnp.int32),
        pltpu.VMEM((N, 4), jnp.float32),
        pltpu.VMEM((CH1,), jnp.int32),
        pltpu.VMEM((CH1,), jnp.int32),
        pltpu.VMEM((CH1,), jnp.int32),
        pltpu.VMEM((4, CH1), jnp.float32),
        pltpu.VMEM((4, CH1), jnp.int32),
        pltpu.VMEM_SHARED((DENP,), jnp.float32),
    ],
    compiler_params=pltpu.CompilerParams(
        needs_layout_passes=False, use_tc_tiling_on_sc=False),
)(_s1_body)


def _s2_body(hcat_h, edge_f, s2_h, ppos_h, pneg_h, den_h, z2_h,
             outp_o, outn_o, bin_o,
             tbl_p, tbl_n, tmp, sbig, dbig, pbig,
             hist_v, curs_v, sb16, poss16, recbuf, recc,
             gb, wb, dlb, rows, accl, histS, gsem, ssem):
    ci = lax.axis_index("c")
    si = lax.axis_index("s")
    iota16 = lax.iota(jnp.int32, 16)
    c625 = jnp.full((16,), RB, jnp.int32)

    def rank_of(b):
        sk, sv = plsc.sort_key_val(b, iota16)
        sb16[pl.ds(0, 16)] = sk
        prevk = plsc.load_gather(sb16, [jnp.maximum(iota16 - 1, 0)])
        nxtk = plsc.load_gather(sb16, [jnp.minimum(iota16 + 1, 15)])
        newseg = (iota16 == 0) | (sk != prevk)
        seglast = (iota16 == 15) | (sk != nxtk)
        segstart = plsc.cummax(jnp.where(newseg, iota16, 0))
        rank = iota16 - segstart
        return sk, sv, rank, seglast

    def load_tbl(tbl, off):
        pltpu.sync_copy(den_h.at[pl.ds(off, N)], tbl)
        for k in range(5):
            pltpu.sync_copy(den_h.at[pl.ds(DENP + off + k * 2000, 2000)], tmp)

            def add16(i, _):
                sl = pl.ds(i * 16, 16)
                tsl = pl.ds(k * 2000 + i * 16, 16)
                tbl[tsl] = tbl[tsl] + tmp[sl]
                return 0
            lax.fori_loop(0, 2000 // 16, add16, 0)

    load_tbl(tbl_p, ci * N)
    load_tbl(tbl_n, (2 + ci) * N)

    hist_v[pl.ds(0, 16)] = jnp.zeros((16,), jnp.int32)

    def count_blk(blk, _):
        base = si * EPT2 + blk * BB
        pltpu.sync_copy(edge_f.at[pl.ds(E + base, BB)], dbig)

        def grp(g, _):
            d16 = dbig[pl.ds(g * 16, 16)]
            b = lax.div(d16, c625)
            sk, _sv, rank, seglast = rank_of(b)
            oldc = plsc.load_gather(hist_v, [sk])
            plsc.store_scatter(hist_v, [sk], oldc + rank + 1, mask=seglast)
            return 0
        lax.fori_loop(0, BB // 16, grp, 0)
        return 0

    lax.fori_loop(0, EPT2 // BB, count_blk, 0)
    pltpu.sync_copy(hist_v, histS.at[si])

    for P in range(2):
        tbl = tbl_p if P == 0 else tbl_n
        p_h = ppos_h if P == 0 else pneg_h
        base_cp = (ci * 2 + P) * (E + CH3)
        hv = hist_v[pl.ds(0, 16)]
        curs_v[pl.ds(0, 16)] = (plsc.cumsum(hv) - hv) + base_cp + si * EPT2

        def bin_blk(blk, _):
            base = si * EPT2 + blk * BB
            pltpu.sync_copy(edge_f.at[pl.ds(E + base, BB)], dbig)
            if P == 0:
                pltpu.sync_copy(edge_f.at[pl.ds(base, BB)], sbig)
            else:
                pltpu.sync_copy(s2_h.at[pl.ds(base, BB)], sbig)
            pltpu.sync_copy(p_h.at[pl.ds(ci * E + base, BB)], pbig)

            def grp(g, _):
                sl = pl.ds(g * 16, 16)
                d16 = dbig[sl]
                s16 = sbig[sl]
                p16 = pbig[sl]
                b = lax.div(d16, c625)
                dstl = d16 - b * RB
                den16 = plsc.load_gather(tbl, [d16])
                alpha = p16 / (den16 + 1e-16)
                hidx = s16 + ci * N
                sk, sv, rank, seglast = rank_of(b)
                cur = plsc.load_gather(curs_v, [sk])
                pos_sorted = cur + rank
                plsc.store_scatter(curs_v, [sk], pos_sorted + 1, mask=seglast)
                plsc.store_scatter(poss16, [sv], pos_sorted)
                pos16 = poss16[pl.ds(0, 16)]
                rid = g * 16 + iota16
                plsc.store_scatter(recbuf, [rid, jnp.zeros((16,), jnp.int32)],
                                   hidx)
                plsc.store_scatter(recbuf, [rid, jnp.full((16,), 1, jnp.int32)],
                                   dstl)
                plsc.store_scatter(recbuf, [rid, jnp.full((16,), 2, jnp.int32)],
                                   plsc.bitcast(alpha, jnp.int32))
                pltpu.make_async_copy(recbuf.at[pl.ds(g * 16, 16)],
                                      bin_o.at[pos16], ssem).start()
                return 0

            def drain(n, _):
                pltpu.make_async_copy(recbuf.at[pl.ds(0, 16)],
                                      bin_o.at[iota16], ssem).wait()
                return 0

            lax.fori_loop(0, BB // 32, grp, 0)
            lax.fori_loop(0, BB // 32, drain, 0)
            lax.fori_loop(BB // 32, BB // 16, grp, 0)
            lax.fori_loop(0, BB // 32, drain, 0)
            return 0

        lax.fori_loop(0, EPT2 // BB, bin_blk, 0)

    plsc.subcore_barrier()

    for P in range(2):
        out_o = outp_o if P == 0 else outn_o
        base_cp = (ci * 2 + P) * (E + CH3)
        pltpu.sync_copy(z2_h, accl)

        def tloop(t, _):
            pltpu.sync_copy(histS.at[t], hist_v)
            hv = hist_v[pl.ds(0, 16)]
            ex = plsc.cumsum(hv) - hv
            msk = iota16 == si
            start_t = jnp.max(jnp.where(msk, ex, 0))
            cnt_t = jnp.max(jnp.where(msk, hv, 0))
            t_base = base_cp + t * EPT2 + start_t

            def chunk(cc, _):
                row0 = t_base + cc * CH3
                pltpu.sync_copy(bin_o.at[pl.ds(row0, CH3)], recc)
                for g in range(CH3 // 16):
                    sl = pl.ds(g * 16, 16)
                    rid = g * 16 + iota16
                    z16 = jnp.zeros((16,), jnp.int32)
                    hidx = plsc.load_gather(recc, [rid, z16])
                    dstl = plsc.load_gather(recc, [rid, z16 + 1])
                    ab = plsc.load_gather(recc, [rid, z16 + 2])
                    valid = (cc * CH3 + rid) < cnt_t
                    gb[sl] = jnp.clip(hidx, 0, 2 * N - 1)
                    dlb[sl] = jnp.clip(dstl, 0, RB - 1)
                    wb[sl] = jnp.where(valid, plsc.bitcast(ab, jnp.float32),
                                       0.0)
                pltpu.async_copy(hcat_h.at[gb], rows, gsem).wait()

                def srow(r, _):
                    rfull = jnp.full((16,), r, jnp.int32)
                    wv = plsc.load_gather(wb, [rfull])
                    dv = plsc.load_gather(dlb, [rfull])
                    for v in range(OUT // 16):
                        col = iota16 + v * 16
                        rv = plsc.load_gather(rows, [rfull, col])
                        plsc.addupdate_scatter(accl, [dv, col], rv * wv)
                    return 0
                lax.fori_loop(0, CH3, srow, 0)
                return 0

            lax.fori_loop(0, (cnt_t + CH3 - 1) // CH3, chunk, 0)
            return 0

        lax.fori_loop(0, NS, tloop, 0)
        pltpu.sync_copy(accl, out_o.at[pl.ds(ci * N + si * RB, RB)])


_stage2 = functools.partial(
    pl.kernel,
    out_type=[
        jax.ShapeDtypeStruct((2 * N, OUT), jnp.float32),
        jax.ShapeDtypeStruct((2 * N, OUT), jnp.float32),
        jax.ShapeDtypeStruct((4 * (E + CH3), 8), jnp.int32),
    ],
    mesh=_mesh,
    scratch_types=[
        pltpu.VMEM((N,), jnp.float32),
        pltpu.VMEM((N,), jnp.float32),
        pltpu.VMEM((2000,), jnp.float32),
        pltpu.VMEM((BB,), jnp.int32),
        pltpu.VMEM((BB,), jnp.int32),
        pltpu.VMEM((BB,), jnp.float32),
        pltpu.VMEM((16,), jnp.int32),
        pltpu.VMEM((16,), jnp.int32),
        pltpu.VMEM((16,), jnp.int32),
        pltpu.VMEM((16,), jnp.int32),
        pltpu.VMEM((BB, 8), jnp.int32),
        pltpu.VMEM((CH3, 8), jnp.int32),
        pltpu.VMEM((CH3,), jnp.int32),
        pltpu.VMEM((CH3,), jnp.float32),
        pltpu.VMEM((CH3,), jnp.int32),
        pltpu.VMEM((CH3, OUT), jnp.float32),
        pltpu.VMEM((RB, OUT), jnp.float32),
        pltpu.VMEM_SHARED((NS, 16), jnp.int32),
        pltpu.SemaphoreType.DMA,
        pltpu.SemaphoreType.DMA,
    ],
    compiler_params=pltpu.CompilerParams(
        needs_layout_passes=False, use_tc_tiling_on_sc=False),
)(_s2_body)


def _ep_body(ph0, ph1, nh0, nh1, b_ref, a_ref, pz, nz, summ, accs):
    i = pl.program_id(0)
    bias = b_ref[...]
    a = a_ref[...]
    z = jnp.concatenate([ph0[...], ph1[...]], axis=1) + bias
    pzb = jnp.where(z > 0, z, a * z)
    pz[...] = pzb
    zn = jnp.concatenate([nh0[...], nh1[...]], axis=1) + bias
    nz[...] = jnp.where(zn > 0, zn, a * zn)

    @pl.when(i == 0)
    def _():
        accs[...] = jnp.zeros_like(accs)
    accs[...] += jnp.sum(pzb, axis=0, keepdims=True)

    @pl.when(i == 4)
    def _():
        summ[...] = jax.nn.sigmoid(accs[...] / N)


def _epilogue(outp, outn, bias, prelu_a):
    bs = 2000
    nb = N // bs
    return pl.pallas_call(
        _ep_body,
        grid=(nb,),
        in_specs=[
            pl.BlockSpec((bs, OUT), lambda i: (i, 0)),
            pl.BlockSpec((bs, OUT), lambda i: (i + nb, 0)),
            pl.BlockSpec((bs, OUT), lambda i: (i, 0)),
            pl.BlockSpec((bs, OUT), lambda i: (i + nb, 0)),
            pl.BlockSpec((1, HO), lambda i: (0, 0)),
            pl.BlockSpec((1, HO), lambda i: (0, 0)),
        ],
        out_specs=[
            pl.BlockSpec((bs, HO), lambda i: (i, 0)),
            pl.BlockSpec((bs, HO), lambda i: (i, 0)),
            pl.BlockSpec((1, HO), lambda i: (0, 0)),
        ],
        out_shape=[
            jax.ShapeDtypeStruct((N, HO), jnp.float32),
            jax.ShapeDtypeStruct((N, HO), jnp.float32),
            jax.ShapeDtypeStruct((1, HO), jnp.float32),
        ],
        scratch_shapes=[pltpu.VMEM((1, HO), jnp.float32)],
    )(outp, outp, outn, outn, bias.reshape(1, HO), prelu_a.reshape(1, HO))


def kernel(x, edge_index, W, att_src, att_dst, bias, prelu_a):
    Waa = jnp.stack([
        W[:, :OUT] @ att_src[0],
        W[:, OUT:] @ att_src[1],
        W[:, :OUT] @ att_dst[0],
        W[:, OUT:] @ att_dst[1],
    ], axis=1)
    perm = jax.random.permutation(jax.random.key(42), N).astype(jnp.int32)
    edge_f = edge_index.reshape(2 * E)

    hcat, aa = _matmul(x, W, Waa)
    zf = jnp.zeros((DENP // NS,), jnp.float32)
    z2 = jnp.zeros((RB, OUT), jnp.float32)
    ppos, pneg, s2, den = _stage1(edge_f, perm, aa, zf)
    outp, outn, _recs = _stage2(hcat, edge_f, s2, ppos, pneg, den, z2)
    pos_z, neg_z, summ = _epilogue(outp, outn, bias, prelu_a)
    return (pos_z, neg_z, summ.reshape(HO))


# speedup vs baseline: 1.1715x; 1.1715x over previous
import functools

import jax
import jax.numpy as jnp
from jax import lax
from jax.experimental import pallas as pl
from jax.experimental.pallas import tpu as pltpu
from jax.experimental.pallas import tpu_sc as plsc

N = 10000
E = 320000
D = 128
HEADS = 2
OUT = 128
HO = HEADS * OUT

NC = 2
NS = 16
NW = NC * NS

EPT1 = E // NW
CH1 = 80
NCH1 = EPT1 // CH1
EPT2 = E // NS
BB = 800
CH3 = 80
DENP = 40960
RB = N // NS
NEG_SLOPE = 0.2

_mesh = plsc.VectorSubcoreMesh(core_axis_name="c", subcore_axis_name="s")


def _mm_body(x_ref, wh_ref, waa_ref, hcat_ref, aa_ref):
    xb = x_ref[...]
    hcat_ref[...] = jnp.dot(xb, wh_ref[...], preferred_element_type=jnp.float32)
    aa_ref[...] = jnp.dot(xb, waa_ref[...], preferred_element_type=jnp.float32)


def _matmul(x, W, Waa):
    bs = 2000
    nb = N // bs
    return pl.pallas_call(
        _mm_body,
        grid=(2 * nb,),
        in_specs=[
            pl.BlockSpec((bs, D), lambda i: (i % nb, 0)),
            pl.BlockSpec((D, OUT), lambda i: (0, i // nb)),
            pl.BlockSpec((D, 4), lambda i: (0, 0)),
        ],
        out_specs=[
            pl.BlockSpec((bs, OUT), lambda i: (i, 0)),
            pl.BlockSpec((bs, 4), lambda i: (i % nb, 0)),
        ],
        out_shape=[
            jax.ShapeDtypeStruct((2 * N, OUT), jnp.float32),
            jax.ShapeDtypeStruct((N, 4), jnp.float32),
        ],
    )(x, W, Waa)


def _s1_body(edge_f, perm_h, aa_h, zf_h,
             ppos_o, pneg_o, s2_o, den_o,
             perm_v, aa_v, sb, db, s2b, p4, vidx, den_acc):
    ci = lax.axis_index("c")
    si = lax.axis_index("s")
    wid = ci * NS + si

    pltpu.sync_copy(perm_h, perm_v)
    pltpu.sync_copy(aa_h, aa_v)
    pltpu.sync_copy(zf_h, den_acc.at[pl.ds(si * (DENP // NS), DENP // NS)])
    plsc.subcore_barrier()

    c0 = jnp.zeros((16,), jnp.int32)
    c1 = jnp.full((16,), 1, jnp.int32)
    c2 = jnp.full((16,), 2, jnp.int32)
    c3 = jnp.full((16,), 3, jnp.int32)

    def lrelu_exp(a, b):
        e = a + b
        return jnp.exp(jnp.where(e > 0, e, NEG_SLOPE * e))

    def chunk(c, _):
        base = wid * EPT1 + c * CH1
        pltpu.sync_copy(edge_f.at[pl.ds(base, CH1)], sb)
        pltpu.sync_copy(edge_f.at[pl.ds(E + base, CH1)], db)
        for g in range(CH1 // 16):
            sl = pl.ds(g * 16, 16)
            s = sb[sl]
            d = db[sl]
            s2 = plsc.load_gather(perm_v, [s])
            d2 = plsc.load_gather(perm_v, [d])
            p0 = lrelu_exp(plsc.load_gather(aa_v, [s, c0]),
                           plsc.load_gather(aa_v, [d, c2]))
            p1 = lrelu_exp(plsc.load_gather(aa_v, [s, c1]),
                           plsc.load_gather(aa_v, [d, c3]))
            q0 = lrelu_exp(plsc.load_gather(aa_v, [s2, c0]),
                           plsc.load_gather(aa_v, [d2, c2]))
            q1 = lrelu_exp(plsc.load_gather(aa_v, [s2, c1]),
                           plsc.load_gather(aa_v, [d2, c3]))
            s2b[sl] = s2
            p4[0, sl] = p0
            p4[1, sl] = p1
            p4[2, sl] = q0
            p4[3, sl] = q1
            vidx[0, sl] = d
            vidx[1, sl] = d + N
            vidx[2, sl] = d + 2 * N
            vidx[3, sl] = d + 3 * N
        pltpu.sync_copy(s2b, s2_o.at[pl.ds(base, CH1)])
        for j in range(4):
            pltpu.sync_copy(p4.at[j], (ppos_o if j < 2 else pneg_o).at[
                pl.ds((j % 2) * E + base, CH1)])
        for j in range(4):
            pltpu.sync_copy(p4.at[j], den_acc.at[vidx.at[j]], add=True)
        return 0

    lax.fori_loop(0, NCH1, chunk, 0)
    plsc.subcore_barrier()
    @pl.when(si < 8)
    def _():
        pltpu.sync_copy(den_acc.at[pl.ds(si * 5120, 5120)],
                        den_o.at[pl.ds(ci * DENP + si * 5120, 5120)])


_stage1 = functools.partial(
    pl.kernel,
    out_type=[
        jax.ShapeDtypeStruct((2 * E,), jnp.float32),
        jax.ShapeDtypeStruct((2 * E,), jnp.float32),
        jax.ShapeDtypeStruct((E,), jnp.int32),
        jax.ShapeDtypeStruct((NC * DENP,), jnp.float32),
    ],
    mesh=_mesh,
    scratch_types=[
        pltpu.VMEM((N,), jnp.int32),
        pltpu.VMEM((N, 4), jnp.float32),
        pltpu.VMEM((CH1,), jnp.int32),
        pltpu.VMEM((CH1,), jnp.int32),
        pltpu.VMEM((CH1,), jnp.int32),
        pltpu.VMEM((4, CH1), jnp.float32),
        pltpu.VMEM((4, CH1), jnp.int32),
        pltpu.VMEM_SHARED((DENP,), jnp.float32),
    ],
    compiler_params=pltpu.CompilerParams(
        needs_layout_passes=False, use_tc_tiling_on_sc=False),
)(_s1_body)


def _s2_body(hcat_h, edge_f, s2_h, ppos_h, pneg_h, den_h, z2_h,
             outp_o, outn_o, bin_o,
             tbl, tmp, sbig, dbig, pbig,
             hist_v, curs_v, sb16, poss16, recbuf,
             rc0, rc1, gb0, gb1, wb0, wb1, dl0, dl1, rw0, rw1,
             accl, histS, gs0, gs1, ssem):
    ci = lax.axis_index("c")
    si = lax.axis_index("s")
    iota16 = lax.iota(jnp.int32, 16)
    c625 = jnp.full((16,), RB, jnp.int32)

    def rank_of(b):
        sk, sv = plsc.sort_key_val(b, iota16)
        sb16[pl.ds(0, 16)] = sk
        prevk = plsc.load_gather(sb16, [jnp.maximum(iota16 - 1, 0)])
        nxtk = plsc.load_gather(sb16, [jnp.minimum(iota16 + 1, 15)])
        newseg = (iota16 == 0) | (sk != prevk)
        seglast = (iota16 == 15) | (sk != nxtk)
        segstart = plsc.cummax(jnp.where(newseg, iota16, 0))
        rank = iota16 - segstart
        return sk, sv, rank, seglast

    def load_tbl(tbl, off):
        pltpu.sync_copy(den_h.at[pl.ds(off, N)], tbl)
        for k in range(5):
            pltpu.sync_copy(den_h.at[pl.ds(DENP + off + k * 2000, 2000)], tmp)

            def add16(i, _):
                sl = pl.ds(i * 16, 16)
                tsl = pl.ds(k * 2000 + i * 16, 16)
                tbl[tsl] = tbl[tsl] + tmp[sl]
                return 0
            lax.fori_loop(0, 2000 // 16, add16, 0)

    hist_v[pl.ds(0, 16)] = jnp.zeros((16,), jnp.int32)

    def count_blk(blk, _):
        base = si * EPT2 + blk * BB
        pltpu.sync_copy(edge_f.at[pl.ds(E + base, BB)], dbig)

        def grp(g, _):
            d16 = dbig[pl.ds(g * 16, 16)]
            b = lax.div(d16, c625)
            sk, _sv, rank, seglast = rank_of(b)
            oldc = plsc.load_gather(hist_v, [sk])
            plsc.store_scatter(hist_v, [sk], oldc + rank + 1, mask=seglast)
            return 0
        lax.fori_loop(0, BB // 16, grp, 0)
        return 0

    lax.fori_loop(0, EPT2 // BB, count_blk, 0)
    pltpu.sync_copy(hist_v, histS.at[si])

    for P in range(2):
        load_tbl(tbl, ci * N if P == 0 else (2 + ci) * N)
        p_h = ppos_h if P == 0 else pneg_h
        base_cp = (ci * 2 + P) * (E + CH3)
        hv = hist_v[pl.ds(0, 16)]
        curs_v[pl.ds(0, 16)] = (plsc.cumsum(hv) - hv) + base_cp + si * EPT2

        def bin_blk(blk, _):
            base = si * EPT2 + blk * BB
            pltpu.sync_copy(edge_f.at[pl.ds(E + base, BB)], dbig)
            if P == 0:
                pltpu.sync_copy(edge_f.at[pl.ds(base, BB)], sbig)
            else:
                pltpu.sync_copy(s2_h.at[pl.ds(base, BB)], sbig)
            pltpu.sync_copy(p_h.at[pl.ds(ci * E + base, BB)], pbig)

            def grp(g, _):
                sl = pl.ds(g * 16, 16)
                d16 = dbig[sl]
                s16 = sbig[sl]
                p16 = pbig[sl]
                b = lax.div(d16, c625)
                dstl = d16 - b * RB
                den16 = plsc.load_gather(tbl, [d16])
                alpha = p16 / (den16 + 1e-16)
                hidx = s16 + ci * N
                sk, sv, rank, seglast = rank_of(b)
                cur = plsc.load_gather(curs_v, [sk])
                pos_sorted = cur + rank
                plsc.store_scatter(curs_v, [sk], pos_sorted + 1, mask=seglast)
                plsc.store_scatter(poss16, [sv], pos_sorted)
                pos16 = poss16[pl.ds(0, 16)]
                rid = g * 16 + iota16
                plsc.store_scatter(recbuf, [rid, jnp.zeros((16,), jnp.int32)],
                                   hidx)
                plsc.store_scatter(recbuf, [rid, jnp.full((16,), 1, jnp.int32)],
                                   dstl)
                plsc.store_scatter(recbuf, [rid, jnp.full((16,), 2, jnp.int32)],
                                   plsc.bitcast(alpha, jnp.int32))
                pltpu.make_async_copy(recbuf.at[pl.ds(g * 16, 16)],
                                      bin_o.at[pos16], ssem).start()
                return 0

            def drain(n, _):
                pltpu.make_async_copy(recbuf.at[pl.ds(0, 16)],
                                      bin_o.at[iota16], ssem).wait()
                return 0

            lax.fori_loop(0, BB // 32, grp, 0)
            lax.fori_loop(0, BB // 32, drain, 0)
            lax.fori_loop(BB // 32, BB // 16, grp, 0)
            lax.fori_loop(0, BB // 32, drain, 0)
            return 0

        lax.fori_loop(0, EPT2 // BB, bin_blk, 0)

    plsc.subcore_barrier()

    for P in range(2):
        out_o = outp_o if P == 0 else outn_o
        base_cp = (ci * 2 + P) * (E + CH3)
        pltpu.sync_copy(z2_h, accl)

        def tloop(t, _):
            pltpu.sync_copy(histS.at[t], hist_v)
            hv = hist_v[pl.ds(0, 16)]
            ex = plsc.cumsum(hv) - hv
            msk = iota16 == si
            start_t = jnp.max(jnp.where(msk, ex, 0))
            cnt_t = jnp.max(jnp.where(msk, hv, 0))
            t_base = base_cp + t * EPT2 + start_t
            nchunk = (cnt_t + CH3 - 1) // CH3

            def fetch(cc, recc, gb, wb, dlb, rows, gsem):
                row0 = t_base + cc * CH3
                pltpu.sync_copy(bin_o.at[pl.ds(row0, CH3)], recc)
                for g in range(CH3 // 16):
                    sl = pl.ds(g * 16, 16)
                    rid = g * 16 + iota16
                    z16 = jnp.zeros((16,), jnp.int32)
                    hidx = plsc.load_gather(recc, [rid, z16])
                    dstl = plsc.load_gather(recc, [rid, z16 + 1])
                    ab = plsc.load_gather(recc, [rid, z16 + 2])
                    valid = (cc * CH3 + rid) < cnt_t
                    gb[sl] = jnp.clip(hidx, 0, 2 * N - 1)
                    dlb[sl] = jnp.clip(dstl, 0, RB - 1)
                    wb[sl] = jnp.where(valid, plsc.bitcast(ab, jnp.float32),
                                       0.0)
                pltpu.make_async_copy(hcat_h.at[gb], rows, gsem).start()

            def compute(wb, dlb, rows):
                def srow(r, _):
                    rfull = jnp.full((16,), r, jnp.int32)
                    wv = plsc.load_gather(wb, [rfull])
                    dv = plsc.load_gather(dlb, [rfull])
                    for v in range(OUT // 16):
                        col = iota16 + v * 16
                        rv = plsc.load_gather(rows, [rfull, col])
                        plsc.addupdate_scatter(accl, [dv, col], rv * wv)
                    return 0
                lax.fori_loop(0, CH3, srow, 0)

            def waitg(gb, rows, gsem):
                pltpu.make_async_copy(hcat_h.at[gb], rows, gsem).wait()

            @pl.when(nchunk > 0)
            def _():
                fetch(0, rc0, gb0, wb0, dl0, rw0, gs0)

            def duo(m, _):
                c0 = 2 * m
                waitg(gb0, rw0, gs0)

                @pl.when(c0 + 1 < nchunk)
                def _():
                    fetch(c0 + 1, rc1, gb1, wb1, dl1, rw1, gs1)
                compute(wb0, dl0, rw0)

                @pl.when(c0 + 1 < nchunk)
                def _():
                    waitg(gb1, rw1, gs1)

                @pl.when(c0 + 2 < nchunk)
                def _():
                    fetch(c0 + 2, rc0, gb0, wb0, dl0, rw0, gs0)

                @pl.when(c0 + 1 < nchunk)
                def _():
                    compute(wb1, dl1, rw1)
                return 0

            lax.fori_loop(0, (nchunk + 1) // 2, duo, 0)
            return 0

        lax.fori_loop(0, NS, tloop, 0)
        pltpu.sync_copy(accl, out_o.at[pl.ds(ci * N + si * RB, RB)])


_stage2 = functools.partial(
    pl.kernel,
    out_type=[
        jax.ShapeDtypeStruct((2 * N, OUT), jnp.float32),
        jax.ShapeDtypeStruct((2 * N, OUT), jnp.float32),
        jax.ShapeDtypeStruct((4 * (E + CH3), 8), jnp.int32),
    ],
    mesh=_mesh,
    scratch_types=[
        pltpu.VMEM((N,), jnp.float32),
        pltpu.VMEM((2000,), jnp.float32),
        pltpu.VMEM((BB,), jnp.int32),
        pltpu.VMEM((BB,), jnp.int32),
        pltpu.VMEM((BB,), jnp.float32),
        pltpu.VMEM((16,), jnp.int32),
        pltpu.VMEM((16,), jnp.int32),
        pltpu.VMEM((16,), jnp.int32),
        pltpu.VMEM((16,), jnp.int32),
        pltpu.VMEM((BB, 8), jnp.int32),
        pltpu.VMEM((CH3, 8), jnp.int32),
        pltpu.VMEM((CH3, 8), jnp.int32),
        pltpu.VMEM((CH3,), jnp.int32),
        pltpu.VMEM((CH3,), jnp.int32),
        pltpu.VMEM((CH3,), jnp.float32),
        pltpu.VMEM((CH3,), jnp.float32),
        pltpu.VMEM((CH3,), jnp.int32),
        pltpu.VMEM((CH3,), jnp.int32),
        pltpu.VMEM((CH3, OUT), jnp.float32),
        pltpu.VMEM((CH3, OUT), jnp.float32),
        pltpu.VMEM((RB, OUT), jnp.float32),
        pltpu.VMEM_SHARED((NS, 16), jnp.int32),
        pltpu.SemaphoreType.DMA,
        pltpu.SemaphoreType.DMA,
        pltpu.SemaphoreType.DMA,
    ],
    compiler_params=pltpu.CompilerParams(
        needs_layout_passes=False, use_tc_tiling_on_sc=False),
)(_s2_body)


def _ep_body(ph0, ph1, nh0, nh1, b_ref, a_ref, pz, nz, summ, accs):
    i = pl.program_id(0)
    bias = b_ref[...]
    a = a_ref[...]
    z = jnp.concatenate([ph0[...], ph1[...]], axis=1) + bias
    pzb = jnp.where(z > 0, z, a * z)
    pz[...] = pzb
    zn = jnp.concatenate([nh0[...], nh1[...]], axis=1) + bias
    nz[...] = jnp.where(zn > 0, zn, a * zn)

    @pl.when(i == 0)
    def _():
        accs[...] = jnp.zeros_like(accs)
    accs[...] += jnp.sum(pzb, axis=0, keepdims=True)

    @pl.when(i == 4)
    def _():
        summ[...] = jax.nn.sigmoid(accs[...] / N)


def _epilogue(outp, outn, bias, prelu_a):
    bs = 2000
    nb = N // bs
    return pl.pallas_call(
        _ep_body,
        grid=(nb,),
        in_specs=[
            pl.BlockSpec((bs, OUT), lambda i: (i, 0)),
            pl.BlockSpec((bs, OUT), lambda i: (i + nb, 0)),
            pl.BlockSpec((bs, OUT), lambda i: (i, 0)),
            pl.BlockSpec((bs, OUT), lambda i: (i + nb, 0)),
            pl.BlockSpec((1, HO), lambda i: (0, 0)),
            pl.BlockSpec((1, HO), lambda i: (0, 0)),
        ],
        out_specs=[
            pl.BlockSpec((bs, HO), lambda i: (i, 0)),
            pl.BlockSpec((bs, HO), lambda i: (i, 0)),
            pl.BlockSpec((1, HO), lambda i: (0, 0)),
        ],
        out_shape=[
            jax.ShapeDtypeStruct((N, HO), jnp.float32),
            jax.ShapeDtypeStruct((N, HO), jnp.float32),
            jax.ShapeDtypeStruct((1, HO), jnp.float32),
        ],
        scratch_shapes=[pltpu.VMEM((1, HO), jnp.float32)],
    )(outp, outp, outn, outn, bias.reshape(1, HO), prelu_a.reshape(1, HO))


def kernel(x, edge_index, W, att_src, att_dst, bias, prelu_a):
    Waa = jnp.stack([
        W[:, :OUT] @ att_src[0],
        W[:, OUT:] @ att_src[1],
        W[:, :OUT] @ att_dst[0],
        W[:, OUT:] @ att_dst[1],
    ], axis=1)
    perm = jax.random.permutation(jax.random.key(42), N).astype(jnp.int32)
    edge_f = edge_index.reshape(2 * E)

    hcat, aa = _matmul(x, W, Waa)
    zf = jnp.zeros((DENP // NS,), jnp.float32)
    z2 = jnp.zeros((RB, OUT), jnp.float32)
    ppos, pneg, s2, den = _stage1(edge_f, perm, aa, zf)
    outp, outn, _recs = _stage2(hcat, edge_f, s2, ppos, pneg, den, z2)
    pos_z, neg_z, summ = _epilogue(outp, outn, bias, prelu_a)
    return (pos_z, neg_z, summ.reshape(HO))
